# baseline (device time: 6780 ns/iter reference)
import jax
import jax.numpy as jnp
from jax import lax
from jax.experimental import pallas as pl
from jax.experimental.pallas import tpu as pltpu

NCHUNK = 2


def kernel(x):
    m, n = x.shape
    rows = m // NCHUNK

    def body(
        x_hbm,
        out_hbm,
        x_vmem,
        send_buf,
        recv_buf,
        out_vmem,
        in_sem,
        out_sems,
        send_sems,
        recv_sems,
    ):
        my_x = lax.axis_index("x")
        my_y = lax.axis_index("y")
        peer = (1 - my_x, my_y)

        barrier = pltpu.get_barrier_semaphore()
        pl.semaphore_signal(
            barrier, inc=1, device_id=peer, device_id_type=pl.DeviceIdType.MESH
        )
        in_dma = pltpu.make_async_copy(x_hbm, x_vmem, in_sem)
        in_dma.start()
        pl.semaphore_wait(barrier, 1)
        in_dma.wait()

        rdmas = []
        for c in range(NCHUNK):
            send_buf[c] = x_vmem[pl.ds(c * rows, rows), :].astype(jnp.bfloat16)
            rdma = pltpu.make_async_remote_copy(
                src_ref=send_buf.at[c],
                dst_ref=recv_buf.at[c],
                send_sem=send_sems.at[c],
                recv_sem=recv_sems.at[c],
                device_id=peer,
                device_id_type=pl.DeviceIdType.MESH,
            )
            rdma.start()
            rdmas.append(rdma)

        out_dmas = []
        for c in range(NCHUNK):
            rdmas[c].wait_recv()
            out_vmem[c] = send_buf[c] + recv_buf[c]
            od = pltpu.make_async_copy(
                out_vmem.at[c],
                out_hbm.at[pl.ds(c * rows, rows), :],
                out_sems.at[c],
            )
            od.start()
            out_dmas.append(od)

        for c in range(NCHUNK):
            out_dmas[c].wait()
            rdmas[c].wait_send()

    return pl.pallas_call(
        body,
        out_shape=jax.ShapeDtypeStruct((m, n), jnp.bfloat16),
        in_specs=[pl.BlockSpec(memory_space=pl.ANY)],
        out_specs=pl.BlockSpec(memory_space=pl.ANY),
        scratch_shapes=[
            pltpu.VMEM((m, n), jnp.float32),
            pltpu.VMEM((NCHUNK, rows, n), jnp.bfloat16),
            pltpu.VMEM((NCHUNK, rows, n), jnp.bfloat16),
            pltpu.VMEM((NCHUNK, rows, n), jnp.bfloat16),
            pltpu.SemaphoreType.DMA,
            pltpu.SemaphoreType.DMA((NCHUNK,)),
            pltpu.SemaphoreType.DMA((NCHUNK,)),
            pltpu.SemaphoreType.DMA((NCHUNK,)),
        ],
        compiler_params=pltpu.CompilerParams(collective_id=0),
    )(x)


# device time: 6659 ns/iter; 1.0182x vs baseline; 1.0182x over previous
import jax
import jax.numpy as jnp
from jax import lax
from jax.experimental import pallas as pl
from jax.experimental.pallas import tpu as pltpu


def kernel(x):
    m, n = x.shape

    def body(x_ref, out_ref, send_buf, recv_buf, send_sem, recv_sem):
        my_x = lax.axis_index("x")
        my_y = lax.axis_index("y")
        peer = (1 - my_x, my_y)

        barrier = pltpu.get_barrier_semaphore()
        pl.semaphore_signal(
            barrier, inc=1, device_id=peer, device_id_type=pl.DeviceIdType.MESH
        )
        send_buf[...] = x_ref[...].astype(jnp.bfloat16)
        pl.semaphore_wait(barrier, 1)

        rdma = pltpu.make_async_remote_copy(
            src_ref=send_buf,
            dst_ref=recv_buf,
            send_sem=send_sem,
            recv_sem=recv_sem,
            device_id=peer,
            device_id_type=pl.DeviceIdType.MESH,
        )
        rdma.start()
        rdma.wait_recv()

        out_ref[...] = send_buf[...] + recv_buf[...]
        rdma.wait_send()

    return pl.pallas_call(
        body,
        out_shape=jax.ShapeDtypeStruct((m, n), jnp.bfloat16),
        in_specs=[pl.BlockSpec(memory_space=pltpu.VMEM)],
        out_specs=pl.BlockSpec(memory_space=pltpu.VMEM),
        scratch_shapes=[
            pltpu.VMEM((m, n), jnp.bfloat16),
            pltpu.VMEM((m, n), jnp.bfloat16),
            pltpu.SemaphoreType.DMA,
            pltpu.SemaphoreType.DMA,
        ],
        compiler_params=pltpu.CompilerParams(collective_id=0),
    )(x)
